# baseline (speedup 1.0000x reference)
import functools

import jax
import jax.numpy as jnp
from jax import lax
from jax.experimental import pallas as pl
from jax.experimental.pallas import tpu as pltpu
from jax.experimental.pallas import tpu_sc as plsc

VOCAB_SIZE = 1000000
EMBED_DIM = 64
NUM_SAMPLED = 64

ROWS_PER_DMA = 128
DMAS_PER_CHUNK = 8
CHUNK = ROWS_PER_DMA * DMAS_PER_CHUNK


@functools.lru_cache(maxsize=None)
def _make_gather(B, D):
    info = plsc.get_sparse_core_info()
    nc, ns = info.num_cores, info.num_subcores
    nw = nc * ns
    assert B % (nw * CHUNK) == 0
    b_per_w = B // nw
    n_chunks = b_per_w // CHUNK

    mesh = plsc.VectorSubcoreMesh(core_axis_name="c", subcore_axis_name="s")

    @functools.partial(
        pl.kernel,
        mesh=mesh,
        out_type=jax.ShapeDtypeStruct((B, D), jnp.float32),
        compiler_params=pltpu.CompilerParams(use_tc_tiling_on_sc=False),
        scratch_types=[
            pltpu.VMEM((DMAS_PER_CHUNK, ROWS_PER_DMA), jnp.int32),
            pltpu.VMEM((CHUNK, D), jnp.float32),
            pltpu.SemaphoreType.DMA,
        ],
    )
    def gather_kernel(table_hbm, idx_hbm, out_hbm, idx_v, rows_v, sem):
        wid = lax.axis_index("s") * nc + lax.axis_index("c")
        base = wid * b_per_w

        def chunk_body(i, carry):
            off = base + i * CHUNK
            pltpu.sync_copy(idx_hbm.at[wid * n_chunks + i], idx_v)
            for j in range(DMAS_PER_CHUNK):
                pltpu.async_copy(
                    table_hbm.at[idx_v.at[j]],
                    rows_v.at[pl.ds(j * ROWS_PER_DMA, ROWS_PER_DMA)],
                    sem,
                )
            for j in range(DMAS_PER_CHUNK):
                pltpu.make_async_copy(
                    table_hbm.at[idx_v.at[j]],
                    rows_v.at[pl.ds(j * ROWS_PER_DMA, ROWS_PER_DMA)],
                    sem,
                ).wait()
            pltpu.sync_copy(rows_v, out_hbm.at[pl.ds(off, CHUNK)])
            return carry

        lax.fori_loop(0, n_chunks, chunk_body, 0, unroll=False)

    return gather_kernel, nw, n_chunks


def kernel(target_index, embedding_table):
    batch = target_index.shape[0]
    skey = jax.random.key(42)
    sampled_idx = jax.random.randint(skey, (batch, NUM_SAMPLED), 1, VOCAB_SIZE)
    B = batch * NUM_SAMPLED
    idx_flat = sampled_idx.reshape(B).astype(jnp.int32)
    gather_kernel, nw, n_chunks = _make_gather(B, EMBED_DIM)
    idx_pages = idx_flat.reshape(nw * n_chunks, DMAS_PER_CHUNK, ROWS_PER_DMA)
    out = gather_kernel(embedding_table, idx_pages)
    return out.reshape(batch, NUM_SAMPLED, EMBED_DIM)


# speedup vs baseline: 1.0211x; 1.0211x over previous
import functools

import jax
import jax.numpy as jnp
from jax import lax
from jax.experimental import pallas as pl
from jax.experimental.pallas import tpu as pltpu
from jax.experimental.pallas import tpu_sc as plsc

VOCAB_SIZE = 1000000
EMBED_DIM = 64
NUM_SAMPLED = 64

ROWS_PER_DMA = 128
DMAS_PER_CHUNK = 4
CHUNK = ROWS_PER_DMA * DMAS_PER_CHUNK


@functools.lru_cache(maxsize=None)
def _make_gather(B, D):
    info = plsc.get_sparse_core_info()
    nc, ns = info.num_cores, info.num_subcores
    nw = nc * ns
    assert B % (nw * CHUNK) == 0
    b_per_w = B // nw
    n_chunks = b_per_w // CHUNK
    idx_rows = b_per_w // ROWS_PER_DMA

    mesh = plsc.VectorSubcoreMesh(core_axis_name="c", subcore_axis_name="s")

    @functools.partial(
        pl.kernel,
        mesh=mesh,
        out_type=jax.ShapeDtypeStruct((B, D), jnp.float32),
        compiler_params=pltpu.CompilerParams(use_tc_tiling_on_sc=False),
        scratch_types=[
            pltpu.VMEM((idx_rows, ROWS_PER_DMA), jnp.int32),
            pltpu.VMEM((CHUNK, D), jnp.float32),
            pltpu.VMEM((CHUNK, D), jnp.float32),
            pltpu.SemaphoreType.DMA,
            pltpu.SemaphoreType.DMA,
        ],
    )
    def gather_kernel(table_hbm, idx_hbm, out_hbm, idx_v, rows0, rows1, g0, g1):
        wid = lax.axis_index("s") * nc + lax.axis_index("c")
        base = wid * b_per_w
        rows = (rows0, rows1)
        gsem = (g0, g1)

        pltpu.sync_copy(idx_hbm.at[wid], idx_v)

        def fire(buf, ci):
            for j in range(DMAS_PER_CHUNK):
                pltpu.async_copy(
                    table_hbm.at[idx_v.at[ci * DMAS_PER_CHUNK + j]],
                    rows[buf].at[pl.ds(j * ROWS_PER_DMA, ROWS_PER_DMA)],
                    gsem[buf],
                )

        def drain(buf):
            pltpu.make_async_copy(
                out_hbm.at[pl.ds(0, CHUNK)], rows[buf], gsem[buf]
            ).wait()

        fire(0, 0)

        def pair_body(p, carry):
            for b in range(2):
                ci = 2 * p + b

                @pl.when(ci + 1 < n_chunks)
                def _():
                    fire(1 - b, ci + 1)

                drain(b)
                pltpu.sync_copy(
                    rows[b], out_hbm.at[pl.ds(base + ci * CHUNK, CHUNK)]
                )
            return carry

        lax.fori_loop(0, n_chunks // 2, pair_body, 0, unroll=False)

    return gather_kernel, nw, idx_rows


def kernel(target_index, embedding_table):
    batch = target_index.shape[0]
    skey = jax.random.key(42)
    sampled_idx = jax.random.randint(skey, (batch, NUM_SAMPLED), 1, VOCAB_SIZE)
    B = batch * NUM_SAMPLED
    idx_flat = sampled_idx.reshape(B).astype(jnp.int32)
    gather_kernel, nw, idx_rows = _make_gather(B, EMBED_DIM)
    idx_pages = idx_flat.reshape(nw, idx_rows, ROWS_PER_DMA)
    out = gather_kernel(embedding_table, idx_pages)
    return out.reshape(batch, NUM_SAMPLED, EMBED_DIM)


# speedup vs baseline: 1.0247x; 1.0035x over previous
import functools

import jax
import jax.numpy as jnp
from jax import lax
from jax.experimental import pallas as pl
from jax.experimental.pallas import tpu as pltpu
from jax.experimental.pallas import tpu_sc as plsc

VOCAB_SIZE = 1000000
EMBED_DIM = 64
NUM_SAMPLED = 64

ROWS_PER_DMA = 128
DMAS_PER_CHUNK = 4
CHUNK = ROWS_PER_DMA * DMAS_PER_CHUNK


@functools.lru_cache(maxsize=None)
def _make_gather(B, D):
    info = plsc.get_sparse_core_info()
    nc, ns = info.num_cores, info.num_subcores
    nw = nc * ns
    assert B % (nw * CHUNK) == 0
    b_per_w = B // nw
    n_chunks = b_per_w // CHUNK
    idx_rows = b_per_w // ROWS_PER_DMA

    mesh = plsc.VectorSubcoreMesh(core_axis_name="c", subcore_axis_name="s")

    @functools.partial(
        pl.kernel,
        mesh=mesh,
        out_type=jax.ShapeDtypeStruct((B, D), jnp.float32),
        compiler_params=pltpu.CompilerParams(use_tc_tiling_on_sc=False),
        scratch_types=[
            pltpu.VMEM((idx_rows, ROWS_PER_DMA), jnp.int32),
            pltpu.VMEM((CHUNK, D), jnp.float32),
            pltpu.VMEM((CHUNK, D), jnp.float32),
            pltpu.SemaphoreType.DMA,
            pltpu.SemaphoreType.DMA,
        ],
    )
    def gather_kernel(table_hbm, idx_hbm, out_hbm, idx_v, rows0, rows1, g0, g1):
        wid = lax.axis_index("s") * nc + lax.axis_index("c")
        base = wid * b_per_w
        rows = (rows0, rows1)
        gsem = (g0, g1)

        pltpu.sync_copy(idx_hbm.at[wid], idx_v)

        def fire(buf, ci):
            for j in range(DMAS_PER_CHUNK):
                pltpu.async_copy(
                    table_hbm.at[idx_v.at[ci * DMAS_PER_CHUNK + j]],
                    rows[buf].at[pl.ds(j * ROWS_PER_DMA, ROWS_PER_DMA)],
                    gsem[buf],
                )

        def drain(buf):
            pltpu.make_async_copy(
                out_hbm.at[pl.ds(0, CHUNK)], rows[buf], gsem[buf]
            ).wait()

        fire(0, 0)

        def pair_body(p, carry):
            for b in range(2):
                ci = 2 * p + b

                @pl.when(ci + 1 < n_chunks)
                def _():
                    fire(1 - b, ci + 1)

                drain(b)
                pltpu.sync_copy(
                    rows[b], out_hbm.at[pl.ds(base + ci * CHUNK, CHUNK)]
                )
            return carry

        lax.fori_loop(0, n_chunks // 2, pair_body, 0, unroll=False)

    return gather_kernel, nw, idx_rows


def kernel(target_index, embedding_table):
    batch = target_index.shape[0]
    B = batch * NUM_SAMPLED
    gather_kernel, nw, idx_rows = _make_gather(B, EMBED_DIM)
    skey = jax.random.key(42)
    idx_pages = jax.random.randint(
        skey, (nw, idx_rows, ROWS_PER_DMA), 1, VOCAB_SIZE, dtype=jnp.int32
    )
    out = gather_kernel(embedding_table, idx_pages)
    return out.reshape(batch, NUM_SAMPLED, EMBED_DIM)
